# baseline (speedup 1.0000x reference)
import functools

import jax
import jax.numpy as jnp
import numpy as np
from jax.experimental import pallas as pl
from jax.experimental.pallas import tpu as pltpu

_QCHUNK = 256


def _hilbert_xy(n: int):
    d = np.arange(n * n, dtype=np.int64)
    x = np.zeros(n * n, dtype=np.int64)
    y = np.zeros(n * n, dtype=np.int64)
    t = d.copy()
    s = 1
    while s < n:
        rx = 1 & (t // 2)
        ry = 1 & (t ^ rx)
        swap = ry == 0
        flip = swap & (rx == 1)
        x = np.where(flip, s - 1 - x, x)
        y = np.where(flip, s - 1 - y, y)
        nx = np.where(swap, y, x)
        ny = np.where(swap, x, y)
        x, y = nx, ny
        x = x + s * rx
        y = y + s * ry
        t = t // 4
        s *= 2
    return x, y


@functools.lru_cache(maxsize=None)
def _quadrant_tables(n: int):
    xs, ys = _hilbert_xy(n)
    nq = n * n // _QCHUNK
    qx, qy = [], []
    onehot = np.zeros((nq, _QCHUNK, _QCHUNK), dtype=np.float32)
    for q in range(nq):
        cx = xs[q * _QCHUNK:(q + 1) * _QCHUNK]
        cy = ys[q * _QCHUNK:(q + 1) * _QCHUNK]
        x0, y0 = int(cx.min()), int(cy.min())
        assert int(cx.max()) - x0 == 15 and int(cy.max()) - y0 == 15
        qx.append(x0 // 16)
        qy.append(y0 // 16)
        src = (cx - x0) * 16 + (cy - y0)
        onehot[q, np.arange(_QCHUNK), src] = 1.0
    return np.asarray(qx, np.int32), np.asarray(qy, np.int32), onehot


def _permute_body(qx_ref, qy_ref, p_ref, x_ref, o_ref):
    block = x_ref[0].reshape(_QCHUNK, x_ref.shape[-1]).astype(jnp.bfloat16)
    o_ref[0, 0] = jnp.dot(p_ref[0], block, preferred_element_type=jnp.float32)


@functools.lru_cache(maxsize=None)
def _make_permute(b: int, n: int, c: int):
    p = n * n
    nq = p // _QCHUNK
    qx, qy, onehot = _quadrant_tables(n)

    grid_spec = pltpu.PrefetchScalarGridSpec(
        num_scalar_prefetch=2,
        grid=(nq, b),
        in_specs=[
            pl.BlockSpec((1, _QCHUNK, _QCHUNK),
                         lambda q, i, qx_ref, qy_ref: (q, 0, 0)),
            pl.BlockSpec((1, 16, 16, c),
                         lambda q, i, qx_ref, qy_ref: (i, qx_ref[q], qy_ref[q], 0)),
        ],
        out_specs=pl.BlockSpec((1, 1, _QCHUNK, c),
                               lambda q, i, qx_ref, qy_ref: (i, 0, q, 0)),
    )
    return pl.pallas_call(
        _permute_body,
        grid_spec=grid_spec,
        out_shape=jax.ShapeDtypeStruct((b, 1, p, c), jnp.float32),
    )


def kernel(inputs):
    b, h, w, c = inputs.shape
    assert h == w
    qx, qy, onehot = _quadrant_tables(h)
    return _make_permute(b, h, c)(
        jnp.asarray(qx), jnp.asarray(qy),
        jnp.asarray(onehot, dtype=jnp.bfloat16), inputs)


# speedup vs baseline: 1.9364x; 1.9364x over previous
import functools

import jax
import jax.numpy as jnp
import numpy as np
from jax.experimental import pallas as pl
from jax.experimental.pallas import tpu as pltpu

_QCHUNK = 256


def _hilbert_xy(n: int):
    d = np.arange(n * n, dtype=np.int64)
    x = np.zeros(n * n, dtype=np.int64)
    y = np.zeros(n * n, dtype=np.int64)
    t = d.copy()
    s = 1
    while s < n:
        rx = 1 & (t // 2)
        ry = 1 & (t ^ rx)
        swap = ry == 0
        flip = swap & (rx == 1)
        x = np.where(flip, s - 1 - x, x)
        y = np.where(flip, s - 1 - y, y)
        nx = np.where(swap, y, x)
        ny = np.where(swap, x, y)
        x, y = nx, ny
        x = x + s * rx
        y = y + s * ry
        t = t // 4
        s *= 2
    return x, y


@functools.lru_cache(maxsize=None)
def _quadrant_tables(n: int):
    xs, ys = _hilbert_xy(n)
    nq = n * n // _QCHUNK
    qx, qy = [], []
    onehot = np.zeros((nq, _QCHUNK, _QCHUNK), dtype=np.float32)
    for q in range(nq):
        cx = xs[q * _QCHUNK:(q + 1) * _QCHUNK]
        cy = ys[q * _QCHUNK:(q + 1) * _QCHUNK]
        x0, y0 = int(cx.min()), int(cy.min())
        assert int(cx.max()) - x0 == 15 and int(cy.max()) - y0 == 15
        qx.append(x0 // 16)
        qy.append(y0 // 16)
        src = (cx - x0) * 16 + (cy - y0)
        onehot[q, np.arange(_QCHUNK), src] = 1.0
    return np.asarray(qx, np.int32), np.asarray(qy, np.int32), onehot


@functools.lru_cache(maxsize=None)
def _make_permute(b: int, n: int, c: int):
    p = n * n
    nq = p // _QCHUNK
    qx, qy, _ = _quadrant_tables(n)

    def body(p_ref, x_ref, o_ref):
        for q in range(nq):
            slab = x_ref[0, 16 * qx[q]:16 * qx[q] + 16,
                         16 * qy[q]:16 * qy[q] + 16, :]
            block = slab.reshape(_QCHUNK, c).astype(jnp.bfloat16)
            o_ref[0, 0, q * _QCHUNK:(q + 1) * _QCHUNK, :] = jnp.dot(
                p_ref[q], block, preferred_element_type=jnp.float32)

    return pl.pallas_call(
        body,
        grid=(b,),
        in_specs=[
            pl.BlockSpec((nq, _QCHUNK, _QCHUNK), lambda i: (0, 0, 0)),
            pl.BlockSpec((1, n, n, c), lambda i: (i, 0, 0, 0)),
        ],
        out_specs=pl.BlockSpec((1, 1, p, c), lambda i: (i, 0, 0, 0)),
        out_shape=jax.ShapeDtypeStruct((b, 1, p, c), jnp.float32),
    )


def kernel(inputs):
    b, h, w, c = inputs.shape
    assert h == w
    _, _, onehot = _quadrant_tables(h)
    return _make_permute(b, h, c)(jnp.asarray(onehot, dtype=jnp.bfloat16), inputs)


# speedup vs baseline: 2.6782x; 1.3831x over previous
import functools

import jax
import jax.numpy as jnp
import numpy as np
from jax.experimental import pallas as pl
from jax.experimental.pallas import tpu as pltpu

_QCHUNK = 256


def _hilbert_xy(n: int):
    d = np.arange(n * n, dtype=np.int64)
    x = np.zeros(n * n, dtype=np.int64)
    y = np.zeros(n * n, dtype=np.int64)
    t = d.copy()
    s = 1
    while s < n:
        rx = 1 & (t // 2)
        ry = 1 & (t ^ rx)
        swap = ry == 0
        flip = swap & (rx == 1)
        x = np.where(flip, s - 1 - x, x)
        y = np.where(flip, s - 1 - y, y)
        nx = np.where(swap, y, x)
        ny = np.where(swap, x, y)
        x, y = nx, ny
        x = x + s * rx
        y = y + s * ry
        t = t // 4
        s *= 2
    return x, y


@functools.lru_cache(maxsize=None)
def _quadrant_tables(n: int):
    xs, ys = _hilbert_xy(n)
    nq = n * n // _QCHUNK
    qx, qy = [], []
    onehot = np.zeros((nq, _QCHUNK, _QCHUNK), dtype=np.float32)
    for q in range(nq):
        cx = xs[q * _QCHUNK:(q + 1) * _QCHUNK]
        cy = ys[q * _QCHUNK:(q + 1) * _QCHUNK]
        x0, y0 = int(cx.min()), int(cy.min())
        assert int(cx.max()) - x0 == 15 and int(cy.max()) - y0 == 15
        qx.append(x0 // 16)
        qy.append(y0 // 16)
        src = (cx - x0) * 16 + (cy - y0)
        onehot[q, np.arange(_QCHUNK), src] = 1.0
    return np.asarray(qx, np.int32), np.asarray(qy, np.int32), onehot


@functools.lru_cache(maxsize=None)
def _make_permute(b: int, n: int, c: int):
    p = n * n
    nq = p // _QCHUNK
    qx, qy, _ = _quadrant_tables(n)

    bblk = 8
    assert b % bblk == 0

    def body(p_ref, x_ref, o_ref):
        for bb in range(bblk):
            for q in range(nq):
                slab = x_ref[bb, 16 * qx[q]:16 * qx[q] + 16,
                             16 * qy[q]:16 * qy[q] + 16, :]
                block = slab.reshape(_QCHUNK, c).astype(jnp.bfloat16)
                o_ref[bb, 0, q * _QCHUNK:(q + 1) * _QCHUNK, :] = jnp.dot(
                    p_ref[q], block, preferred_element_type=jnp.float32)

    return pl.pallas_call(
        body,
        grid=(b // bblk,),
        in_specs=[
            pl.BlockSpec((nq, _QCHUNK, _QCHUNK), lambda i: (0, 0, 0)),
            pl.BlockSpec((bblk, n, n, c), lambda i: (i, 0, 0, 0)),
        ],
        out_specs=pl.BlockSpec((bblk, 1, p, c), lambda i: (i, 0, 0, 0)),
        out_shape=jax.ShapeDtypeStruct((b, 1, p, c), jnp.float32),
    )


def kernel(inputs):
    b, h, w, c = inputs.shape
    assert h == w
    _, _, onehot = _quadrant_tables(h)
    return _make_permute(b, h, c)(jnp.asarray(onehot, dtype=jnp.bfloat16), inputs)


# speedup vs baseline: 4.0133x; 1.4985x over previous
import functools

import jax
import jax.numpy as jnp
import numpy as np
from jax.experimental import pallas as pl

_QCHUNK = 256


def _hilbert_xy(n: int):
    d = np.arange(n * n, dtype=np.int64)
    x = np.zeros(n * n, dtype=np.int64)
    y = np.zeros(n * n, dtype=np.int64)
    t = d.copy()
    s = 1
    while s < n:
        rx = 1 & (t // 2)
        ry = 1 & (t ^ rx)
        swap = ry == 0
        flip = swap & (rx == 1)
        x = np.where(flip, s - 1 - x, x)
        y = np.where(flip, s - 1 - y, y)
        nx = np.where(swap, y, x)
        ny = np.where(swap, x, y)
        x, y = nx, ny
        x = x + s * rx
        y = y + s * ry
        t = t // 4
        s *= 2
    return x, y


@functools.lru_cache(maxsize=None)
def _quadrant_tables(n: int):
    xs, ys = _hilbert_xy(n)
    nq = n * n // _QCHUNK
    qx, qy = [], []
    onehot = np.zeros((nq, _QCHUNK, _QCHUNK), dtype=np.float32)
    for q in range(nq):
        cx = xs[q * _QCHUNK:(q + 1) * _QCHUNK]
        cy = ys[q * _QCHUNK:(q + 1) * _QCHUNK]
        x0, y0 = int(cx.min()), int(cy.min())
        assert int(cx.max()) - x0 == 15 and int(cy.max()) - y0 == 15
        qx.append(x0 // 16)
        qy.append(y0 // 16)
        src = (cx - x0) * 16 + (cy - y0)
        onehot[q, np.arange(_QCHUNK), src] = 1.0
    return np.asarray(qx, np.int32), np.asarray(qy, np.int32), onehot


@functools.lru_cache(maxsize=None)
def _make_permute(b: int, n: int, c: int):
    p = n * n
    nq = p // _QCHUNK
    qx_np, qy_np, _ = _quadrant_tables(n)
    assert [int(v) for v in qx_np] == [q // 2 for q in range(nq)]
    assert [int(v) for v in qy_np] == [(q // 2) ^ (q % 2) for q in range(nq)]
    ch = c // 2

    def body(q_ref, i_ref, x_ref, o_ref):
        x = x_ref[...].reshape(_QCHUNK, ch * b).astype(jnp.bfloat16)
        g = jnp.dot(q_ref[0], x, preferred_element_type=jnp.float32)
        m = g.reshape(_QCHUNK * ch, b).astype(jnp.bfloat16)
        t = jax.lax.dot_general(i_ref[...], m, (((1,), (1,)), ((), ())),
                                preferred_element_type=jnp.float32)
        v = t.reshape(b, _QCHUNK, ch)
        o_ref[...] = jnp.transpose(v, (0, 2, 1)).reshape(b, 1, ch, _QCHUNK)

    call = pl.pallas_call(
        body,
        grid=(nq * 2,),
        in_specs=[
            pl.BlockSpec((1, _QCHUNK, _QCHUNK), lambda s: (s // 2, 0, 0)),
            pl.BlockSpec((b, b), lambda s: (0, 0)),
            pl.BlockSpec((16, 16, ch, b),
                         lambda s: (s // 4, (s // 4) ^ ((s // 2) % 2), s % 2, 0)),
        ],
        out_specs=pl.BlockSpec((b, 1, ch, _QCHUNK),
                               lambda s: (0, 0, s % 2, s // 2)),
        out_shape=jax.ShapeDtypeStruct((b, 1, c, p), jnp.float32),
    )
    return call


def kernel(inputs):
    b, h, w, c = inputs.shape
    assert h == w
    _, _, onehot = _quadrant_tables(h)
    x2 = jnp.transpose(inputs, (1, 2, 3, 0))
    out2 = _make_permute(b, h, c)(
        jnp.asarray(onehot, jnp.bfloat16),
        jnp.eye(b, dtype=jnp.bfloat16), x2)
    return jnp.transpose(out2, (0, 1, 3, 2))


# speedup vs baseline: 11.3407x; 2.8258x over previous
import functools

import jax
import jax.numpy as jnp
import numpy as np
from jax.experimental import pallas as pl

_QCHUNK = 256


def _hilbert_xy(n: int):
    d = np.arange(n * n, dtype=np.int64)
    x = np.zeros(n * n, dtype=np.int64)
    y = np.zeros(n * n, dtype=np.int64)
    t = d.copy()
    s = 1
    while s < n:
        rx = 1 & (t // 2)
        ry = 1 & (t ^ rx)
        swap = ry == 0
        flip = swap & (rx == 1)
        x = np.where(flip, s - 1 - x, x)
        y = np.where(flip, s - 1 - y, y)
        nx = np.where(swap, y, x)
        ny = np.where(swap, x, y)
        x, y = nx, ny
        x = x + s * rx
        y = y + s * ry
        t = t // 4
        s *= 2
    return x, y


@functools.lru_cache(maxsize=None)
def _quadrant_tables(n: int):
    xs, ys = _hilbert_xy(n)
    nq = n * n // _QCHUNK
    qx, qy = [], []
    onehot = np.zeros((nq, _QCHUNK, _QCHUNK), dtype=np.float32)
    for q in range(nq):
        cx = xs[q * _QCHUNK:(q + 1) * _QCHUNK]
        cy = ys[q * _QCHUNK:(q + 1) * _QCHUNK]
        x0, y0 = int(cx.min()), int(cy.min())
        assert int(cx.max()) - x0 == 15 and int(cy.max()) - y0 == 15
        qx.append(x0 // 16)
        qy.append(y0 // 16)
        src = (cx - x0) * 16 + (cy - y0)
        onehot[q, np.arange(_QCHUNK), src] = 1.0
    return np.asarray(qx, np.int32), np.asarray(qy, np.int32), onehot


@functools.lru_cache(maxsize=None)
def _make_permute(b: int, n: int, c: int):
    p = n * n
    nq = p // _QCHUNK
    qx_np, qy_np, _ = _quadrant_tables(n)
    assert [int(v) for v in qx_np] == [q // 2 for q in range(nq)]
    assert [int(v) for v in qy_np] == [(q // 2) ^ (q % 2) for q in range(nq)]
    ch = c // 2

    def body(q_ref, x_ref, o_ref):
        x3 = x_ref[...].reshape(_QCHUNK, ch, b).astype(jnp.bfloat16)
        g = jax.lax.dot_general(x3, q_ref[0], (((0,), (1,)), ((), ())),
                                preferred_element_type=jnp.float32)
        o_ref[...] = jnp.transpose(g, (1, 0, 2)).reshape(b, 1, ch, _QCHUNK)

    call = pl.pallas_call(
        body,
        grid=(nq * 2,),
        in_specs=[
            pl.BlockSpec((1, _QCHUNK, _QCHUNK), lambda s: (s // 2, 0, 0)),
            pl.BlockSpec((16, 16, ch, b),
                         lambda s: (s // 4, (s // 4) ^ ((s // 2) % 2), s % 2, 0)),
        ],
        out_specs=pl.BlockSpec((b, 1, ch, _QCHUNK),
                               lambda s: (0, 0, s % 2, s // 2)),
        out_shape=jax.ShapeDtypeStruct((b, 1, c, p), jnp.float32),
    )
    return call


def kernel(inputs):
    b, h, w, c = inputs.shape
    assert h == w
    _, _, onehot = _quadrant_tables(h)
    x2 = jnp.transpose(inputs, (1, 2, 3, 0))
    out2 = _make_permute(b, h, c)(jnp.asarray(onehot, jnp.bfloat16), x2)
    return jnp.transpose(out2, (0, 1, 3, 2))
